# baseline (speedup 1.0000x reference)
import functools

import jax
import jax.numpy as jnp
from jax import lax
from jax.experimental import pallas as pl
from jax.experimental.pallas import tpu as pltpu
from jax.experimental.pallas import tpu_sc as plsc

NUM_CORES = 2
NUM_SUBCORES = 16
NW = NUM_CORES * NUM_SUBCORES
CHUNK = 128
EMB_DIM = 64
NBUF = 8
K = 4

RB = 128
PAIR = 2 * EMB_DIM


def _make_repack(vocab: int):
    n_blocks = (vocab + RB - 1) // RB
    vocab_pad = n_blocks * RB
    n_loop = (n_blocks + NW - 1) // NW
    mesh = plsc.VectorSubcoreMesh(core_axis_name="c", subcore_axis_name="s")

    @functools.partial(
        pl.kernel,
        mesh=mesh,
        out_type=jax.ShapeDtypeStruct((vocab_pad // 2, PAIR), jnp.float32),
        scratch_types=[
            pltpu.VMEM((2, EMB_DIM, RB), jnp.float32),
            pltpu.VMEM((2, RB // 2, PAIR), jnp.float32),
            pltpu.SemaphoreType.DMA((2,)),
            pltpu.SemaphoreType.DMA((2,)),
        ],
        compiler_params=pltpu.CompilerParams(
            use_tc_tiling_on_sc=True, needs_layout_passes=False
        ),
    )
    def repack(t2_hbm, r_hbm, a_v, b_v, asem, bsem):
        wid = lax.axis_index("s") * NUM_CORES + lax.axis_index("c")
        rows = [jax.lax.iota(jnp.int32, 16) + 16 * q for q in range(4)]

        pltpu.async_copy(
            t2_hbm.at[:, pl.ds(wid * RB, RB)], a_v.at[0], asem.at[0]
        )

        def body(k, carry):
            vb = wid + NW * k
            slot = lax.rem(k, 2)
            pvb = vb - 2 * NW

            @pl.when(jnp.logical_and(k >= 2, pvb < n_blocks))
            def _():
                pltpu.make_async_copy(
                    b_v.at[slot],
                    r_hbm.at[pl.ds(pvb * (RB // 2), RB // 2)],
                    bsem.at[slot],
                ).wait()

            @pl.when(vb < n_blocks)
            def _():
                pltpu.make_async_copy(
                    t2_hbm.at[:, pl.ds(vb * RB, RB)], a_v.at[slot], asem.at[slot]
                ).wait()

                nvb = vb + NW

                @pl.when(nvb < n_blocks)
                def _():
                    pltpu.async_copy(
                        t2_hbm.at[:, pl.ds(nvb * RB, RB)],
                        a_v.at[1 - slot],
                        asem.at[1 - slot],
                    )

                def permute(s_loc, c2):
                    col0 = jnp.full((16,), 2 * s_loc, jnp.int32)
                    col1 = col0 + 1
                    for c in range(8):
                        vals = plsc.load_gather(
                            a_v.at[slot],
                            [rows[c % 4], col0 if c < 4 else col1],
                        )
                        b_v[slot, s_loc, pl.ds(c * 16, 16)] = vals
                    return c2

                lax.fori_loop(0, RB // 2, permute, 0)
                pltpu.async_copy(
                    b_v.at[slot],
                    r_hbm.at[pl.ds(vb * (RB // 2), RB // 2)],
                    bsem.at[slot],
                )

            return carry

        lax.fori_loop(0, n_loop, body, 0)

        for s in range(2):
            kk = n_loop - 2 + s
            vb = wid + NW * kk

            @pl.when(vb < n_blocks)
            def _():
                pltpu.make_async_copy(
                    b_v.at[kk % 2],
                    r_hbm.at[pl.ds(vb * (RB // 2), RB // 2)],
                    bsem.at[kk % 2],
                ).wait()

    return repack


def _make_lookup(n_chunks: int, n_rows: int):
    assert n_chunks % NBUF == 0 and n_chunks > NBUF
    mesh = plsc.VectorSubcoreMesh(core_axis_name="c", subcore_axis_name="s")

    @functools.partial(
        pl.kernel,
        mesh=mesh,
        out_type=jax.ShapeDtypeStruct((NW, n_chunks, CHUNK, EMB_DIM), jnp.float32),
        scratch_types=[
            pltpu.VMEM((n_chunks, CHUNK), jnp.int32),
            pltpu.VMEM((NBUF, CHUNK, EMB_DIM), jnp.float32),
            pltpu.SemaphoreType.DMA((NBUF,)),
            pltpu.SemaphoreType.DMA((NBUF,)),
        ],
        compiler_params=pltpu.CompilerParams(use_tc_tiling_on_sc=False),
    )
    def lookup(idx_hbm, table_hbm, out_hbm, idx_v, buf, gsem, wsem):
        wid = lax.axis_index("s") * NUM_CORES + lax.axis_index("c")
        pltpu.sync_copy(idx_hbm.at[wid], idx_v)

        for b in range(K):
            pltpu.async_copy(table_hbm.at[idx_v.at[b]], buf.at[b], gsem.at[b])

        def body(j, carry):
            nj = j + K
            ns = lax.rem(nj, NBUF)

            @pl.when(jnp.logical_and(nj < n_chunks, nj >= NBUF))
            def _():
                pltpu.make_async_copy(
                    buf.at[ns], out_hbm.at[wid, nj - NBUF], wsem.at[ns]
                ).wait()

            @pl.when(nj < n_chunks)
            def _():
                pltpu.async_copy(table_hbm.at[idx_v.at[nj]], buf.at[ns], gsem.at[ns])

            b = lax.rem(j, NBUF)
            pltpu.make_async_copy(
                table_hbm.at[idx_v.at[j]], buf.at[b], gsem.at[b]
            ).wait()
            pltpu.async_copy(buf.at[b], out_hbm.at[wid, j], wsem.at[b])
            return carry

        lax.fori_loop(0, n_chunks, body, 0)

        for b in range(NBUF):
            pltpu.make_async_copy(
                buf.at[b], out_hbm.at[wid, n_chunks - NBUF + b], wsem.at[b]
            ).wait()

    return lookup


def kernel(x, table):
    vocab, emb = table.shape
    b, s = x.shape[1], x.shape[2]
    total = b * s
    n_chunks = total // (NW * CHUNK)

    t2 = jnp.transpose(table)
    r = _make_repack(vocab)(t2)
    rows = jnp.reshape(r, (r.shape[0] * 2, emb))

    idx = jnp.reshape(x[0].astype(jnp.int32), (NW, n_chunks, CHUNK))
    out = _make_lookup(n_chunks, rows.shape[0])(idx, rows)
    return jnp.reshape(out, (b, s, EMB_DIM))


# speedup vs baseline: 1.7106x; 1.7106x over previous
import functools

import jax
import jax.numpy as jnp
from jax import lax
from jax.experimental import pallas as pl
from jax.experimental.pallas import tpu as pltpu
from jax.experimental.pallas import tpu_sc as plsc

NUM_CORES = 2
NUM_SUBCORES = 16
NW = NUM_CORES * NUM_SUBCORES
CHUNK = 128
EMB_DIM = 64
NBUF = 8
K = 4


def _make_lookup(n_chunks: int):
    assert n_chunks % NBUF == 0 and n_chunks > NBUF
    mesh = plsc.VectorSubcoreMesh(core_axis_name="c", subcore_axis_name="s")

    @functools.partial(
        pl.kernel,
        mesh=mesh,
        out_type=jax.ShapeDtypeStruct((NW, n_chunks, CHUNK, EMB_DIM), jnp.float32),
        scratch_types=[
            pltpu.VMEM((n_chunks, CHUNK), jnp.int32),
            pltpu.VMEM((NBUF, CHUNK, EMB_DIM), jnp.float32),
            pltpu.SemaphoreType.DMA((NBUF,)),
            pltpu.SemaphoreType.DMA((NBUF,)),
        ],
        compiler_params=pltpu.CompilerParams(use_tc_tiling_on_sc=False),
    )
    def lookup(idx_hbm, table_hbm, out_hbm, idx_v, buf, gsem, wsem):
        wid = lax.axis_index("s") * NUM_CORES + lax.axis_index("c")
        pltpu.sync_copy(idx_hbm.at[wid], idx_v)

        for b in range(K):
            pltpu.async_copy(table_hbm.at[idx_v.at[b]], buf.at[b], gsem.at[b])

        def body(j, carry):
            nj = j + K
            ns = lax.rem(nj, NBUF)

            @pl.when(jnp.logical_and(nj < n_chunks, nj >= NBUF))
            def _():
                pltpu.make_async_copy(
                    buf.at[ns], out_hbm.at[wid, nj - NBUF], wsem.at[ns]
                ).wait()

            @pl.when(nj < n_chunks)
            def _():
                pltpu.async_copy(table_hbm.at[idx_v.at[nj]], buf.at[ns], gsem.at[ns])

            b = lax.rem(j, NBUF)
            pltpu.make_async_copy(
                table_hbm.at[idx_v.at[j]], buf.at[b], gsem.at[b]
            ).wait()
            pltpu.async_copy(buf.at[b], out_hbm.at[wid, j], wsem.at[b])
            return carry

        lax.fori_loop(0, n_chunks, body, 0)

        for b in range(NBUF):
            pltpu.make_async_copy(
                buf.at[b], out_hbm.at[wid, n_chunks - NBUF + b], wsem.at[b]
            ).wait()

    return lookup


def kernel(x, table):
    b, s = x.shape[1], x.shape[2]
    total = b * s
    n_chunks = total // (NW * CHUNK)
    idx = jnp.reshape(x[0].astype(jnp.int32), (NW, n_chunks, CHUNK))
    out = _make_lookup(n_chunks)(idx, table)
    return jnp.reshape(out, (b, s, EMB_DIM))


# speedup vs baseline: 1.7153x; 1.0028x over previous
import functools

import jax
import jax.numpy as jnp
from jax import lax
from jax.experimental import pallas as pl
from jax.experimental.pallas import tpu as pltpu
from jax.experimental.pallas import tpu_sc as plsc

NUM_CORES = 2
NUM_SUBCORES = 16
NW = NUM_CORES * NUM_SUBCORES
CHUNK = 128
EMB_DIM = 64
NBUF = 8
K = 6


def _make_lookup(n_chunks: int):
    assert n_chunks % NBUF == 0 and n_chunks > NBUF
    mesh = plsc.VectorSubcoreMesh(core_axis_name="c", subcore_axis_name="s")

    @functools.partial(
        pl.kernel,
        mesh=mesh,
        out_type=jax.ShapeDtypeStruct((NW, n_chunks, CHUNK, EMB_DIM), jnp.float32),
        scratch_types=[
            pltpu.VMEM((n_chunks, CHUNK), jnp.int32),
            pltpu.VMEM((NBUF, CHUNK, EMB_DIM), jnp.float32),
            pltpu.SemaphoreType.DMA((NBUF,)),
            pltpu.SemaphoreType.DMA((NBUF,)),
        ],
        compiler_params=pltpu.CompilerParams(use_tc_tiling_on_sc=False),
    )
    def lookup(idx_hbm, table_hbm, out_hbm, idx_v, buf, gsem, wsem):
        wid = lax.axis_index("s") * NUM_CORES + lax.axis_index("c")
        pltpu.sync_copy(idx_hbm.at[wid], idx_v)

        for b in range(K):
            pltpu.async_copy(table_hbm.at[idx_v.at[b]], buf.at[b], gsem.at[b])

        def body(j, carry):
            nj = j + K
            ns = lax.rem(nj, NBUF)

            @pl.when(jnp.logical_and(nj < n_chunks, nj >= NBUF))
            def _():
                pltpu.make_async_copy(
                    buf.at[ns], out_hbm.at[wid, nj - NBUF], wsem.at[ns]
                ).wait()

            @pl.when(nj < n_chunks)
            def _():
                pltpu.async_copy(table_hbm.at[idx_v.at[nj]], buf.at[ns], gsem.at[ns])

            b = lax.rem(j, NBUF)
            pltpu.make_async_copy(
                table_hbm.at[idx_v.at[j]], buf.at[b], gsem.at[b]
            ).wait()
            pltpu.async_copy(buf.at[b], out_hbm.at[wid, j], wsem.at[b])
            return carry

        lax.fori_loop(0, n_chunks, body, 0)

        for b in range(NBUF):
            pltpu.make_async_copy(
                buf.at[b], out_hbm.at[wid, n_chunks - NBUF + b], wsem.at[b]
            ).wait()

    return lookup


def kernel(x, table):
    b, s = x.shape[1], x.shape[2]
    total = b * s
    n_chunks = total // (NW * CHUNK)
    idx = jnp.reshape(x[0].astype(jnp.int32), (NW, n_chunks, CHUNK))
    out = _make_lookup(n_chunks)(idx, table)
    return jnp.reshape(out, (b, s, EMB_DIM))
